# Initial kernel scaffold; baseline (speedup 1.0000x reference)
import jax
import jax.numpy as jnp
from jax.experimental import pallas as pl


def kernel(indices, table):
    raise NotImplementedError("write your pallas kernel here")


# speedup vs baseline: 1.8411x; 1.8411x over previous
import functools

import jax
import jax.numpy as jnp
from jax import lax
from jax.experimental import pallas as pl
from jax.experimental.pallas import tpu as pltpu
from jax.experimental.pallas import tpu_sc as plsc


@functools.lru_cache(maxsize=None)
def _make_gather(B, V, D, dtype_name):
    dtype = jnp.dtype(dtype_name)
    info = plsc.get_sparse_core_info()
    NW = info.num_cores * info.num_subcores
    NC = info.num_cores
    assert B % NW == 0
    b_per_w = B // NW
    chunk = 640
    while b_per_w % chunk != 0:
        chunk //= 2
    nchunk = b_per_w // chunk
    assert chunk % 8 == 0 and b_per_w % 8 == 0

    mesh = plsc.VectorSubcoreMesh(core_axis_name="c", subcore_axis_name="s")

    @functools.partial(
        pl.kernel,
        mesh=mesh,
        out_type=jax.ShapeDtypeStruct((B, D), dtype),
        scratch_types=[
            pltpu.VMEM((b_per_w,), jnp.int32),
            pltpu.VMEM((chunk, D), dtype),
            pltpu.SemaphoreType.DMA,
        ],
        compiler_params=pltpu.CompilerParams(use_tc_tiling_on_sc=False),
    )
    def gather_kernel(idx_hbm, table_hbm, out_hbm, idx_v, rows_v, sem):
        wid = lax.axis_index("s") * NC + lax.axis_index("c")
        base = wid * b_per_w
        pltpu.sync_copy(idx_hbm.at[pl.ds(base, b_per_w)], idx_v)

        def chunk_body(g, carry):
            off = g * chunk
            pltpu.async_copy(
                table_hbm.at[idx_v.at[pl.ds(off, chunk)]], rows_v, sem
            ).wait()
            pltpu.sync_copy(rows_v, out_hbm.at[pl.ds(base + off, chunk)])
            return carry

        lax.fori_loop(0, nchunk, chunk_body, 0)

    return gather_kernel


def kernel(indices, table):
    BATCH, HIST = indices.shape
    V, D = table.shape
    idx = indices.reshape(-1).astype(jnp.int32)
    gather = _make_gather(idx.shape[0], V, D, str(table.dtype))
    out = gather(idx, table)
    return out.reshape(BATCH, HIST, D)


# speedup vs baseline: 1.8767x; 1.0193x over previous
import functools

import jax
import jax.numpy as jnp
from jax import lax
from jax.experimental import pallas as pl
from jax.experimental.pallas import tpu as pltpu
from jax.experimental.pallas import tpu_sc as plsc


@functools.lru_cache(maxsize=None)
def _make_gather(B, V, D, dtype_name):
    dtype = jnp.dtype(dtype_name)
    info = plsc.get_sparse_core_info()
    NW = info.num_cores * info.num_subcores
    NC = info.num_cores
    assert B % NW == 0
    b_per_w = B // NW
    nbuf = 4
    chunk = 320
    while b_per_w % (chunk * nbuf) != 0:
        chunk //= 2
    nchunk = b_per_w // chunk
    assert chunk % 8 == 0 and b_per_w % 8 == 0

    mesh = plsc.VectorSubcoreMesh(core_axis_name="c", subcore_axis_name="s")

    @functools.partial(
        pl.kernel,
        mesh=mesh,
        out_type=jax.ShapeDtypeStruct((B, D), dtype),
        scratch_types=[
            pltpu.VMEM((b_per_w,), jnp.int32),
            pltpu.VMEM((nbuf, chunk, D), dtype),
            pltpu.SemaphoreType.DMA((nbuf,)),
        ],
        compiler_params=pltpu.CompilerParams(use_tc_tiling_on_sc=False),
    )
    def gather_kernel(idx_hbm, table_hbm, out_hbm, idx_v, rows_v, gsem):
        wid = lax.axis_index("s") * NC + lax.axis_index("c")
        base = wid * b_per_w
        pltpu.sync_copy(idx_hbm.at[pl.ds(base, b_per_w)], idx_v)

        def gather_chunk(g, b):
            return pltpu.make_async_copy(
                table_hbm.at[idx_v.at[pl.ds(g * chunk, chunk)]],
                rows_v.at[b],
                gsem.at[b],
            )

        for b in range(nbuf):
            gather_chunk(b, b).start()

        def outer(g0, carry):
            for b in range(nbuf):
                g = g0 * nbuf + b
                gather_chunk(g, b).wait()
                pltpu.sync_copy(
                    rows_v.at[b], out_hbm.at[pl.ds(base + g * chunk, chunk)]
                )

                @pl.when(g + nbuf < nchunk)
                def _():
                    gather_chunk(g + nbuf, b).start()

            return carry

        lax.fori_loop(0, nchunk // nbuf, outer, 0)

    return gather_kernel


def kernel(indices, table):
    BATCH, HIST = indices.shape
    V, D = table.shape
    idx = indices.reshape(-1).astype(jnp.int32)
    gather = _make_gather(idx.shape[0], V, D, str(table.dtype))
    out = gather(idx, table)
    return out.reshape(BATCH, HIST, D)


# speedup vs baseline: 2.2588x; 1.2036x over previous
import functools

import jax
import jax.numpy as jnp
from jax import lax
from jax.experimental import pallas as pl
from jax.experimental.pallas import tpu as pltpu
from jax.experimental.pallas import tpu_sc as plsc


@functools.lru_cache(maxsize=None)
def _make_gather(BATCH, HIST, V, D, dtype_name):
    dtype = jnp.dtype(dtype_name)
    info = plsc.get_sparse_core_info()
    NW = info.num_cores * info.num_subcores
    NC = info.num_cores
    L = 128
    SUB = 8
    assert D % SUB == 0 and BATCH % (L * NW) == 0
    DT = D // SUB
    HT = (HIST + SUB - 1) // SUB
    KB = BATCH // (L * NW)
    ROWS = L * KB
    PITCH = L + 1

    mesh = plsc.VectorSubcoreMesh(core_axis_name="c", subcore_axis_name="s")

    @functools.partial(
        pl.kernel,
        mesh=mesh,
        out_type=jax.ShapeDtypeStruct((HIST, DT, BATCH // L, SUB, L), dtype),
        scratch_types=[
            pltpu.VMEM((ROWS, HIST), jnp.int32),
            pltpu.VMEM((HIST, ROWS), jnp.int32),
            pltpu.VMEM((KB, L, D), dtype),
            pltpu.VMEM((KB, D, PITCH), dtype),
            pltpu.SemaphoreType.DMA((KB,)),
            pltpu.SemaphoreType.DMA((KB,)),
            pltpu.SemaphoreType.DMA,
        ],
        compiler_params=pltpu.CompilerParams(
            use_tc_tiling_on_sc=False, needs_layout_passes=False
        ),
    )
    def gather_kernel(idx_hbm, table_hbm, out_hbm, idx_s, idx_t, gbuf, tbuf,
                      gsem, osem, ssem):
        wid = lax.axis_index("s") * NC + lax.axis_index("c")
        base = wid * ROWS
        pltpu.async_copy(
            idx_hbm.at[pl.ds(base, ROWS)], idx_s, ssem,
        ).wait()

        iota = lax.iota(jnp.int32, 16)

        col_starts = []
        c = 0
        while c + 16 <= HIST:
            col_starts.append(c)
            c += 16
        if col_starts[-1] + 16 < HIST:
            col_starts.append(HIST - 16)

        def idx_t_body(r, carry):
            col = jnp.full((16,), r, jnp.int32)
            for c0 in col_starts:
                v = idx_s[r, pl.ds(c0, 16)]
                plsc.store_scatter(idx_t, [c0 + iota, col], v)
            return carry

        lax.fori_loop(0, ROWS, idx_t_body, 0)

        def gather_block(h, k):
            return pltpu.make_async_copy(
                table_hbm.at[idx_t.at[h, pl.ds(L * k, L)]],
                gbuf.at[k],
                gsem.at[k],
            )

        def out_copy(h, k, dt):
            bt = KB * wid + k
            return pltpu.make_async_copy(
                tbuf.at[k, pl.ds(SUB * dt, SUB), pl.ds(0, L)],
                out_hbm.at[h, dt, bt],
                osem.at[k],
            )

        for k in range(KB):
            gather_block(0, k).start()

        row_ids = [16 * c + iota for c in range(D // 16)]

        def h_body(h, carry):
            for k in range(KB):
                gather_block(h, k).wait()

                @pl.when(h > 0)
                def _():
                    for dt in range(DT):
                        out_copy(h - 1, k, dt).wait()

                def t_body(bl, carry2):
                    col = jnp.full((16,), bl, jnp.int32)
                    for c in range(D // 16):
                        v = gbuf[k, bl, pl.ds(16 * c, 16)]
                        plsc.store_scatter(
                            tbuf.at[k], [row_ids[c], col], v
                        )
                    return carry2

                lax.fori_loop(0, L, t_body, 0)

                @pl.when(h + 1 < HIST)
                def _():
                    gather_block(h + 1, k).start()

                for dt in range(DT):
                    out_copy(h, k, dt).start()
            return carry

        lax.fori_loop(0, HIST, h_body, 0)

        for k in range(KB):
            for dt in range(DT):
                out_copy(HIST - 1, k, dt).wait()

    return gather_kernel


def kernel(indices, table):
    BATCH, HIST = indices.shape
    V, D = table.shape
    idx = indices.astype(jnp.int32)
    gather = _make_gather(BATCH, HIST, V, D, str(table.dtype))
    out5 = gather(idx, table)
    return out5.transpose(2, 4, 0, 1, 3).reshape(BATCH, HIST, D)


# speedup vs baseline: 3.0577x; 1.3537x over previous
import functools

import jax
import jax.numpy as jnp
from jax import lax
from jax.experimental import pallas as pl
from jax.experimental.pallas import tpu as pltpu
from jax.experimental.pallas import tpu_sc as plsc


@functools.lru_cache(maxsize=None)
def _make_gather(BATCH, HIST, V, D, dtype_name):
    dtype = jnp.dtype(dtype_name)
    info = plsc.get_sparse_core_info()
    NW = info.num_cores * info.num_subcores
    NC = info.num_cores
    L = 128
    SUB = 8
    assert D % SUB == 0 and BATCH % (L * NW) == 0
    DT = D // SUB
    HT = (HIST + SUB - 1) // SUB
    KB = BATCH // (L * NW)
    ROWS = L * KB
    PITCH = L + 1

    mesh = plsc.VectorSubcoreMesh(core_axis_name="c", subcore_axis_name="s")

    @functools.partial(
        pl.kernel,
        mesh=mesh,
        out_type=jax.ShapeDtypeStruct((HIST, DT, BATCH // L, SUB, L), dtype),
        scratch_types=[
            pltpu.VMEM((ROWS * HIST,), jnp.int32),
            pltpu.VMEM((HIST, ROWS), jnp.int32),
            pltpu.VMEM((KB, L, D), dtype),
            pltpu.VMEM((KB, D, PITCH), dtype),
            pltpu.SemaphoreType.DMA((KB,)),
            pltpu.SemaphoreType.DMA((KB,)),
            pltpu.SemaphoreType.DMA,
        ],
        compiler_params=pltpu.CompilerParams(
            use_tc_tiling_on_sc=False, needs_layout_passes=False
        ),
    )
    def gather_kernel(idx_hbm, table_hbm, out_hbm, idx_s, idx_t, gbuf, tbuf,
                      gsem, osem, ssem):
        wid = lax.axis_index("s") * NC + lax.axis_index("c")
        base = wid * ROWS
        pltpu.async_copy(
            idx_hbm.at[pl.ds(base * HIST, ROWS * HIST)], idx_s, ssem,
        ).wait()

        iota = lax.iota(jnp.int32, 16)

        col_starts = []
        c = 0
        while c + 16 <= HIST:
            col_starts.append(c)
            c += 16
        if col_starts[-1] + 16 < HIST:
            col_starts.append(HIST - 16)

        @plsc.parallel_loop(0, ROWS, 1, unroll=8)
        def idx_t_body(r):
            col = jnp.full((16,), r, jnp.int32)
            for c0 in col_starts:
                v = idx_s[pl.ds(r * HIST + c0, 16)]
                plsc.store_scatter(idx_t, [c0 + iota, col], v)

        def gather_block(h, k):
            return pltpu.make_async_copy(
                table_hbm.at[idx_t.at[h, pl.ds(L * k, L)]],
                gbuf.at[k],
                gsem.at[k],
            )

        def out_copy(h, k, dt):
            bt = KB * wid + k
            return pltpu.make_async_copy(
                tbuf.at[k, pl.ds(SUB * dt, SUB), pl.ds(0, L)],
                out_hbm.at[h, dt, bt],
                osem.at[k],
            )

        for k in range(KB):
            gather_block(0, k).start()

        row_ids = [16 * c + iota for c in range(D // 16)]

        def h_body(h, carry):
            for k in range(KB):
                gather_block(h, k).wait()

                @pl.when(h > 0)
                def _():
                    for dt in range(DT):
                        out_copy(h - 1, k, dt).wait()

                @plsc.parallel_loop(0, L, 1, unroll=8)
                def t_body(bl):
                    col = jnp.full((16,), bl, jnp.int32)
                    for c in range(D // 16):
                        v = gbuf[k, bl, pl.ds(16 * c, 16)]
                        plsc.store_scatter(
                            tbuf.at[k], [row_ids[c], col], v
                        )

                @pl.when(h + 1 < HIST)
                def _():
                    gather_block(h + 1, k).start()

                for dt in range(DT):
                    out_copy(h, k, dt).start()
            return carry

        lax.fori_loop(0, HIST, h_body, 0)

        for k in range(KB):
            for dt in range(DT):
                out_copy(HIST - 1, k, dt).wait()

    return gather_kernel


def kernel(indices, table):
    BATCH, HIST = indices.shape
    V, D = table.shape
    idx = indices.reshape(-1).astype(jnp.int32)
    gather = _make_gather(BATCH, HIST, V, D, str(table.dtype))
    out5 = gather(idx, table)
    return out5.transpose(2, 4, 0, 1, 3).reshape(BATCH, HIST, D)


# speedup vs baseline: 3.1297x; 1.0235x over previous
import functools

import jax
import jax.numpy as jnp
from jax import lax
from jax.experimental import pallas as pl
from jax.experimental.pallas import tpu as pltpu
from jax.experimental.pallas import tpu_sc as plsc


@functools.lru_cache(maxsize=None)
def _make_gather(BATCH, HIST, V, D, dtype_name):
    dtype = jnp.dtype(dtype_name)
    info = plsc.get_sparse_core_info()
    NW = info.num_cores * info.num_subcores
    NC = info.num_cores
    L = 128
    SUB = 8
    assert D % SUB == 0 and BATCH % (L * NW) == 0
    DT = D // SUB
    HT = (HIST + SUB - 1) // SUB
    KB = BATCH // (L * NW)
    ROWS = L * KB
    PITCH = L + 1

    mesh = plsc.VectorSubcoreMesh(core_axis_name="c", subcore_axis_name="s")

    @functools.partial(
        pl.kernel,
        mesh=mesh,
        out_type=jax.ShapeDtypeStruct((HIST, DT, BATCH // L, SUB, L), dtype),
        scratch_types=[
            pltpu.VMEM((KB, HIST, L), jnp.int32),
            pltpu.VMEM((KB, L, D), dtype),
            pltpu.VMEM((KB, D, PITCH), dtype),
            pltpu.SemaphoreType.DMA((KB,)),
            pltpu.SemaphoreType.DMA((KB,)),
            pltpu.SemaphoreType.DMA,
        ],
        compiler_params=pltpu.CompilerParams(
            use_tc_tiling_on_sc=False, needs_layout_passes=False
        ),
    )
    def gather_kernel(idx_hbm, table_hbm, out_hbm, islab, gbuf, tbuf,
                      gsem, osem, ssem):
        wid = lax.axis_index("s") * NC + lax.axis_index("c")

        iota = lax.iota(jnp.int32, 16)

        for k in range(KB):
            pltpu.async_copy(
                idx_hbm.at[:, pl.ds(L * (KB * wid + k), L)],
                islab.at[k], ssem,
            ).wait()

        def gather_block(h, k):
            return pltpu.make_async_copy(
                table_hbm.at[islab.at[k, h]],
                gbuf.at[k],
                gsem.at[k],
            )

        def out_copy(h, k, dt):
            bt = KB * wid + k
            return pltpu.make_async_copy(
                tbuf.at[k, pl.ds(SUB * dt, SUB), pl.ds(0, L)],
                out_hbm.at[h, dt, bt],
                osem.at[k],
            )

        for k in range(KB):
            gather_block(0, k).start()

        row_ids = [16 * c + iota for c in range(D // 16)]

        def h_body(h, carry):
            for k in range(KB):
                gather_block(h, k).wait()

                @pl.when(h > 0)
                def _():
                    for dt in range(DT):
                        out_copy(h - 1, k, dt).wait()

                @plsc.parallel_loop(0, L, 1, unroll=8)
                def t_body(bl):
                    col = jnp.full((16,), bl, jnp.int32)
                    for c in range(D // 16):
                        v = gbuf[k, bl, pl.ds(16 * c, 16)]
                        plsc.store_scatter(
                            tbuf.at[k], [row_ids[c], col], v
                        )

                @pl.when(h + 1 < HIST)
                def _():
                    gather_block(h + 1, k).start()

                for dt in range(DT):
                    out_copy(h, k, dt).start()
            return carry

        lax.fori_loop(0, HIST, h_body, 0)

        for k in range(KB):
            for dt in range(DT):
                out_copy(HIST - 1, k, dt).wait()

    return gather_kernel


def kernel(indices, table):
    BATCH, HIST = indices.shape
    V, D = table.shape
    idx_t = indices.astype(jnp.int32).T
    gather = _make_gather(BATCH, HIST, V, D, str(table.dtype))
    out5 = gather(idx_t, table)
    return out5.transpose(2, 4, 0, 1, 3).reshape(BATCH, HIST, D)
